# Initial kernel scaffold; baseline (speedup 1.0000x reference)
import jax
import jax.numpy as jnp
from jax.experimental import pallas as pl


def kernel(student_out, teacher_out, codebook, teacher_codes):
    raise NotImplementedError("write your pallas kernel here")


# speedup vs baseline: 7.2978x; 7.2978x over previous
import functools

import jax
import jax.numpy as jnp
from jax import lax
from jax.experimental import pallas as pl
from jax.experimental.pallas import tpu as pltpu
from jax.experimental.pallas import tpu_sc as plsc

MARGIN_ = 0.5


def _argmin_body(z_ref, cb_ref, tc_ref, idx_ref, *, kk):
    zb = z_ref[...]
    cb = cb_ref[...]
    zc = lax.dot_general(zb, cb, (((1,), (1,)), ((), ())),
                         preferred_element_type=jnp.float32)
    z2 = jnp.sum(zb * zb, axis=1, keepdims=True)
    c2 = jnp.sum(cb * cb, axis=1)[None, :]
    d2 = jnp.maximum(z2 + c2 - 2.0 * zc, 1e-12)
    col = lax.broadcasted_iota(jnp.int32, d2.shape, 1)
    tcb = tc_ref[...].reshape(-1, 1)
    d2m = jnp.where(col == tcb, jnp.inf, d2)
    m = jnp.min(d2m, axis=1, keepdims=True)
    idx = jnp.min(jnp.where(d2m == m, col, kk), axis=1)
    idx_ref[...] = idx.reshape(1, 1, -1)


def _hard_neg_idx(z, codebook, tc):
    n, c = z.shape
    kk = codebook.shape[0]
    tb = 256
    n_tb = n // tb
    tc3 = tc.reshape(n_tb, 1, tb)
    idx3 = pl.pallas_call(
        functools.partial(_argmin_body, kk=kk),
        grid=(n_tb,),
        in_specs=[
            pl.BlockSpec((tb, c), lambda i: (i, 0)),
            pl.BlockSpec((kk, c), lambda i: (0, 0)),
            pl.BlockSpec((1, 1, tb), lambda i: (i, 0, 0)),
        ],
        out_specs=pl.BlockSpec((1, 1, tb), lambda i: (i, 0, 0)),
        out_shape=jax.ShapeDtypeStruct((n_tb, 1, tb), jnp.int32),
    )(z, codebook, tc3)
    return idx3.reshape(n)


def _sc_gather(codebook, idx):
    n = idx.shape[0]
    d = codebook.shape[1]
    info = plsc.get_sparse_core_info()
    nc, ns = info.num_cores, info.num_subcores
    nw = nc * ns
    b_per_w = n // nw
    mesh = plsc.VectorSubcoreMesh(core_axis_name="c", subcore_axis_name="s")

    @functools.partial(
        pl.kernel, mesh=mesh,
        out_type=jax.ShapeDtypeStruct((n, d), jnp.float32),
        scratch_types=[
            pltpu.VMEM((b_per_w,), jnp.int32),
            pltpu.VMEM((b_per_w, d), jnp.float32),
            pltpu.SemaphoreType.DMA,
        ],
    )
    def gather_k(table_hbm, idx_hbm, out_hbm, idx_v, rows_v, sem):
        wid = lax.axis_index("s") * nc + lax.axis_index("c")
        base = wid * b_per_w
        pltpu.sync_copy(idx_hbm.at[pl.ds(base, b_per_w)], idx_v)
        pltpu.async_copy(table_hbm.at[idx_v], rows_v, sem).wait()
        pltpu.sync_copy(rows_v, out_hbm.at[pl.ds(base, b_per_w)])

    return gather_k(codebook, idx)


def _loss_body(z_ref, t_ref, n_ref, out_ref, *, n_total, n_blocks):
    i = pl.program_id(0)
    zb = z_ref[...]
    tb = t_ref[...]
    nb = n_ref[...]
    dp = jnp.sqrt(jnp.maximum(jnp.sum((tb - zb) ** 2, axis=1), 1e-12))
    dn = jnp.sqrt(jnp.maximum(jnp.sum((tb - nb) ** 2, axis=1), 1e-12))
    losses = jnp.maximum(dp - dn + MARGIN_, 0.0)
    sat = (dn > dp + MARGIN_).astype(jnp.float32)
    row = lax.broadcasted_iota(jnp.int32, (8, 128), 0)
    part = jnp.where(row == 0, jnp.sum(losses),
           jnp.where(row == 1, jnp.sum(dp),
           jnp.where(row == 2, jnp.sum(dn),
           jnp.where(row == 3, jnp.sum(sat), 0.0))))

    @pl.when(i == 0)
    def _():
        out_ref[...] = jnp.zeros_like(out_ref)

    out_ref[...] += part

    @pl.when(i == n_blocks - 1)
    def _():
        out_ref[...] = out_ref[...] / float(n_total)


def _triplet_stats(z, t, negs):
    n, c = z.shape
    nb = 512
    n_blocks = n // nb
    out = pl.pallas_call(
        functools.partial(_loss_body, n_total=n, n_blocks=n_blocks),
        grid=(n_blocks,),
        in_specs=[
            pl.BlockSpec((nb, c), lambda i: (i, 0)),
            pl.BlockSpec((nb, c), lambda i: (i, 0)),
            pl.BlockSpec((nb, c), lambda i: (i, 0)),
        ],
        out_specs=pl.BlockSpec((8, 128), lambda i: (0, 0)),
        out_shape=jax.ShapeDtypeStruct((8, 128), jnp.float32),
    )(z, t, negs)
    return out[0, 0], out[1, 0], out[2, 0], out[3, 0]


def kernel(student_out, teacher_out, codebook, teacher_codes):
    b, c, t = student_out.shape
    n = b * t
    z = jnp.transpose(student_out, (0, 2, 1)).reshape(n, c)
    tt = jnp.transpose(teacher_out, (0, 2, 1)).reshape(n, c)
    tc = teacher_codes.astype(jnp.int32).reshape(n)

    idx = _hard_neg_idx(z, codebook, tc)
    negs = _sc_gather(codebook, idx)
    loss, d_pos, d_neg, sat = _triplet_stats(z, tt, negs)
    return (loss, d_pos, d_neg, sat)


# speedup vs baseline: 8.4366x; 1.1560x over previous
import functools

import jax
import jax.numpy as jnp
from jax import lax
from jax.experimental import pallas as pl
from jax.experimental.pallas import tpu as pltpu
from jax.experimental.pallas import tpu_sc as plsc

MARGIN_ = 0.5


def _argmin_body(s_ref, t_ref, tc_ref, cb_ref, idx_ref, dp_ref, c2_ref, *, kk):
    @pl.when(pl.program_id(0) == 0)
    def _():
        cb0 = cb_ref[...]
        ones = jnp.ones((cb0.shape[1], 1), jnp.float32)
        c2_ref[...] = lax.dot_general(cb0 * cb0, ones, (((1,), (0,)), ((), ())),
                                      preferred_element_type=jnp.float32)

    sb = s_ref[0]
    zc = lax.dot_general(cb_ref[...], sb * (-2.0), (((1,), (0,)), ((), ())),
                         preferred_element_type=jnp.float32)
    z2 = jnp.sum(sb * sb, axis=0)[None, :]
    e = jnp.maximum(zc + c2_ref[...], 1e-12 - z2)
    row = lax.broadcasted_iota(jnp.int32, e.shape, 0)
    tcb = tc_ref[...].reshape(1, -1)
    em = jnp.where(row == tcb, jnp.inf, e)
    m = jnp.min(em, axis=0, keepdims=True)
    idx = jnp.min(jnp.where(em == m, row, kk), axis=0)
    idx_ref[...] = idx.reshape(1, 1, -1)

    tb = t_ref[0]
    diff = tb - sb
    dp2 = jnp.sum(diff * diff, axis=0)
    dp_ref[...] = jnp.sqrt(jnp.maximum(dp2, 1e-12)).reshape(1, 1, -1)


def _mine_and_dpos(student_out, teacher_out, codebook, tc3):
    b, c, t = student_out.shape
    kk = codebook.shape[0]
    idx3, dpos3 = pl.pallas_call(
        functools.partial(_argmin_body, kk=kk),
        grid=(b,),
        in_specs=[
            pl.BlockSpec((1, c, t), lambda i: (i, 0, 0)),
            pl.BlockSpec((1, c, t), lambda i: (i, 0, 0)),
            pl.BlockSpec((1, 1, t), lambda i: (i, 0, 0)),
            pl.BlockSpec((kk, c), lambda i: (0, 0)),
        ],
        out_specs=[
            pl.BlockSpec((1, 1, t), lambda i: (i, 0, 0)),
            pl.BlockSpec((1, 1, t), lambda i: (i, 0, 0)),
        ],
        out_shape=[
            jax.ShapeDtypeStruct((b, 1, t), jnp.int32),
            jax.ShapeDtypeStruct((b, 1, t), jnp.float32),
        ],
        scratch_shapes=[pltpu.VMEM((kk, 1), jnp.float32)],
    )(student_out, teacher_out, tc3, codebook)
    return idx3.reshape(b * t), dpos3


def _sc_gather(codebook, idx):
    n = idx.shape[0]
    d = codebook.shape[1]
    info = plsc.get_sparse_core_info()
    nc, ns = info.num_cores, info.num_subcores
    nw = nc * ns
    b_per_w = n // nw
    mesh = plsc.VectorSubcoreMesh(core_axis_name="c", subcore_axis_name="s")

    @functools.partial(
        pl.kernel, mesh=mesh,
        out_type=jax.ShapeDtypeStruct((n, d), jnp.float32),
        scratch_types=[
            pltpu.VMEM((b_per_w,), jnp.int32),
            pltpu.VMEM((b_per_w, d), jnp.float32),
            pltpu.SemaphoreType.DMA,
        ],
    )
    def gather_k(table_hbm, idx_hbm, out_hbm, idx_v, rows_v, sem):
        wid = lax.axis_index("s") * nc + lax.axis_index("c")
        base = wid * b_per_w
        pltpu.sync_copy(idx_hbm.at[pl.ds(base, b_per_w)], idx_v)
        pltpu.async_copy(table_hbm.at[idx_v], rows_v, sem).wait()
        pltpu.sync_copy(rows_v, out_hbm.at[pl.ds(base, b_per_w)])

    return gather_k(codebook, idx)


def _loss_body(t_ref, n_ref, dp_ref, out_ref, *, n_total, n_blocks):
    i = pl.program_id(0)
    tb = t_ref[0]
    nb = jnp.transpose(n_ref[0], (1, 0))
    diff = tb - nb
    dn = jnp.sqrt(jnp.maximum(jnp.sum(diff * diff, axis=0), 1e-12))
    dp = dp_ref[...].reshape(-1)
    losses = jnp.maximum(dp - dn + MARGIN_, 0.0)
    sat = (dn > dp + MARGIN_).astype(jnp.float32)
    lane = lax.broadcasted_iota(jnp.int32, (8, 128), 0)
    part = jnp.where(lane == 0, jnp.sum(losses),
           jnp.where(lane == 1, jnp.sum(dp),
           jnp.where(lane == 2, jnp.sum(dn),
           jnp.where(lane == 3, jnp.sum(sat), 0.0))))

    @pl.when(i == 0)
    def _():
        out_ref[...] = jnp.zeros_like(out_ref)

    out_ref[...] += part

    @pl.when(i == n_blocks - 1)
    def _():
        out_ref[...] = out_ref[...] / float(n_total)


def _triplet_stats(teacher_out, negs3, dpos3):
    b, c, t = teacher_out.shape
    out = pl.pallas_call(
        functools.partial(_loss_body, n_total=b * t, n_blocks=b),
        grid=(b,),
        in_specs=[
            pl.BlockSpec((1, c, t), lambda i: (i, 0, 0)),
            pl.BlockSpec((1, t, c), lambda i: (i, 0, 0)),
            pl.BlockSpec((1, 1, t), lambda i: (i, 0, 0)),
        ],
        out_specs=pl.BlockSpec((8, 128), lambda i: (0, 0)),
        out_shape=jax.ShapeDtypeStruct((8, 128), jnp.float32),
    )(teacher_out, negs3, dpos3)
    return out[0, 0], out[1, 0], out[2, 0], out[3, 0]


def kernel(student_out, teacher_out, codebook, teacher_codes):
    b, c, t = student_out.shape
    tc3 = teacher_codes.astype(jnp.int32).reshape(b, 1, t)
    idx, dpos3 = _mine_and_dpos(student_out, teacher_out, codebook, tc3)
    negs = _sc_gather(codebook, idx)
    loss, d_pos, d_neg, sat = _triplet_stats(
        teacher_out, negs.reshape(b, t, c), dpos3)
    return (loss, d_pos, d_neg, sat)


# speedup vs baseline: 8.8384x; 1.0476x over previous
import functools

import jax
import jax.numpy as jnp
from jax import lax
from jax.experimental import pallas as pl
from jax.experimental.pallas import tpu as pltpu
from jax.experimental.pallas import tpu_sc as plsc

MARGIN_ = 0.5


def _argmin_body(z_ref, t_ref, cb_ref, tc_ref, idx_ref, dp_ref, c2_ref, *, kk):
    @pl.when(pl.program_id(0) == 0)
    def _():
        cb0 = cb_ref[...]
        ones = jnp.ones((1, cb0.shape[1]), jnp.float32)
        c2_ref[...] = lax.dot_general(ones, cb0 * cb0, (((1,), (1,)), ((), ())),
                                      preferred_element_type=jnp.float32)

    zb = z_ref[...]
    zc = lax.dot_general(zb * (-2.0), cb_ref[...], (((1,), (1,)), ((), ())),
                         preferred_element_type=jnp.float32)
    z2 = jnp.sum(zb * zb, axis=1, keepdims=True)
    e = jnp.maximum(zc + c2_ref[...], 1e-12 - z2)
    col = lax.broadcasted_iota(jnp.int32, e.shape, 1)
    tcb = tc_ref[...].reshape(-1, 1)
    em = jnp.where(col == tcb, jnp.inf, e)
    m = jnp.min(em, axis=1, keepdims=True)
    idx = jnp.min(jnp.where(em == m, col, kk), axis=1)
    idx_ref[...] = idx.reshape(1, 1, -1)

    tb = t_ref[...]
    diff = tb - zb
    dp2 = jnp.sum(diff * diff, axis=1)
    dp_ref[...] = jnp.sqrt(jnp.maximum(dp2, 1e-12)).reshape(1, 1, -1)


def _mine_and_dpos(z, t, codebook, tc):
    n, c = z.shape
    kk = codebook.shape[0]
    tb = 256
    n_tb = n // tb
    tc3 = tc.reshape(n_tb, 1, tb)
    idx3, dpos3 = pl.pallas_call(
        functools.partial(_argmin_body, kk=kk),
        grid=(n_tb,),
        in_specs=[
            pl.BlockSpec((tb, c), lambda i: (i, 0)),
            pl.BlockSpec((tb, c), lambda i: (i, 0)),
            pl.BlockSpec((kk, c), lambda i: (0, 0)),
            pl.BlockSpec((1, 1, tb), lambda i: (i, 0, 0)),
        ],
        out_specs=[
            pl.BlockSpec((1, 1, tb), lambda i: (i, 0, 0)),
            pl.BlockSpec((1, 1, tb), lambda i: (i, 0, 0)),
        ],
        out_shape=[
            jax.ShapeDtypeStruct((n_tb, 1, tb), jnp.int32),
            jax.ShapeDtypeStruct((n_tb, 1, tb), jnp.float32),
        ],
        scratch_shapes=[pltpu.VMEM((1, kk), jnp.float32)],
    )(z, t, codebook, tc3)
    return idx3.reshape(n), dpos3


def _sc_gather(codebook, idx):
    n = idx.shape[0]
    d = codebook.shape[1]
    info = plsc.get_sparse_core_info()
    nc, ns = info.num_cores, info.num_subcores
    nw = nc * ns
    b_per_w = n // nw
    mesh = plsc.VectorSubcoreMesh(core_axis_name="c", subcore_axis_name="s")

    @functools.partial(
        pl.kernel, mesh=mesh,
        out_type=jax.ShapeDtypeStruct((n, d), jnp.float32),
        scratch_types=[
            pltpu.VMEM((b_per_w,), jnp.int32),
            pltpu.VMEM((b_per_w, d), jnp.float32),
            pltpu.SemaphoreType.DMA,
        ],
    )
    def gather_k(table_hbm, idx_hbm, out_hbm, idx_v, rows_v, sem):
        wid = lax.axis_index("s") * nc + lax.axis_index("c")
        base = wid * b_per_w
        pltpu.sync_copy(idx_hbm.at[pl.ds(base, b_per_w)], idx_v)
        pltpu.async_copy(table_hbm.at[idx_v], rows_v, sem).wait()
        pltpu.sync_copy(rows_v, out_hbm.at[pl.ds(base, b_per_w)])

    return gather_k(codebook, idx)


def _loss_body(t_ref, n_ref, dp_ref, out_ref, *, n_total, n_blocks):
    i = pl.program_id(0)
    tb = t_ref[...]
    nb = n_ref[...]
    dn = jnp.sqrt(jnp.maximum(jnp.sum((tb - nb) ** 2, axis=1), 1e-12))
    dp = dp_ref[...].reshape(-1)
    losses = jnp.maximum(dp - dn + MARGIN_, 0.0)
    sat = (dn > dp + MARGIN_).astype(jnp.float32)
    lane = lax.broadcasted_iota(jnp.int32, (8, 128), 0)
    part = jnp.where(lane == 0, jnp.sum(losses),
           jnp.where(lane == 1, jnp.sum(dp),
           jnp.where(lane == 2, jnp.sum(dn),
           jnp.where(lane == 3, jnp.sum(sat), 0.0))))

    @pl.when(i == 0)
    def _():
        out_ref[...] = jnp.zeros_like(out_ref)

    out_ref[...] += part

    @pl.when(i == n_blocks - 1)
    def _():
        out_ref[...] = out_ref[...] / float(n_total)


def _triplet_stats(t, negs, dpos3):
    n, c = t.shape
    nb = 512
    n_blocks = n // nb
    dp2 = dpos3.reshape(n_blocks, 1, nb)
    out = pl.pallas_call(
        functools.partial(_loss_body, n_total=n, n_blocks=n_blocks),
        grid=(n_blocks,),
        in_specs=[
            pl.BlockSpec((nb, c), lambda i: (i, 0)),
            pl.BlockSpec((nb, c), lambda i: (i, 0)),
            pl.BlockSpec((1, 1, nb), lambda i: (i, 0, 0)),
        ],
        out_specs=pl.BlockSpec((8, 128), lambda i: (0, 0)),
        out_shape=jax.ShapeDtypeStruct((8, 128), jnp.float32),
    )(t, negs, dp2)
    return out[0, 0], out[1, 0], out[2, 0], out[3, 0]


def kernel(student_out, teacher_out, codebook, teacher_codes):
    b, c, t = student_out.shape
    n = b * t
    z = jnp.transpose(student_out, (0, 2, 1)).reshape(n, c)
    tt = jnp.transpose(teacher_out, (0, 2, 1)).reshape(n, c)
    tc = teacher_codes.astype(jnp.int32).reshape(n)

    idx, dpos3 = _mine_and_dpos(z, tt, codebook, tc)
    negs = _sc_gather(codebook, idx)
    loss, d_pos, d_neg, sat = _triplet_stats(tt, negs, dpos3)
    return (loss, d_pos, d_neg, sat)


# speedup vs baseline: 9.1897x; 1.0397x over previous
import functools

import jax
import jax.numpy as jnp
from jax import lax
from jax.experimental import pallas as pl
from jax.experimental.pallas import tpu as pltpu
from jax.experimental.pallas import tpu_sc as plsc

MARGIN_ = 0.5


def _argmin_body(z_ref, t_ref, cb_ref, tc_ref, idx_ref, dp_ref, c2_ref, *, kk):
    @pl.when(pl.program_id(0) == 0)
    def _():
        cb0 = cb_ref[...]
        ones = jnp.ones((1, cb0.shape[1]), jnp.float32)
        c2_ref[...] = lax.dot_general(ones, cb0 * cb0, (((1,), (1,)), ((), ())),
                                      preferred_element_type=jnp.float32)

    zb = z_ref[...]
    zc = lax.dot_general(zb * (-2.0), cb_ref[...], (((1,), (1,)), ((), ())),
                         preferred_element_type=jnp.float32)
    z2 = jnp.sum(zb * zb, axis=1, keepdims=True)
    e = jnp.maximum(zc + c2_ref[...], 1e-12 - z2)
    col = lax.broadcasted_iota(jnp.int32, e.shape, 1)
    tcb = tc_ref[...].reshape(-1, 1)
    em = jnp.where(col == tcb, jnp.inf, e)
    idx = jnp.argmin(em, axis=1).astype(jnp.int32)
    idx_ref[...] = idx.reshape(1, 1, -1)

    tb = t_ref[...]
    diff = tb - zb
    dp2 = jnp.sum(diff * diff, axis=1)
    dp_ref[...] = jnp.sqrt(jnp.maximum(dp2, 1e-12)).reshape(1, 1, -1)


def _mine_and_dpos(z, t, codebook, tc):
    n, c = z.shape
    kk = codebook.shape[0]
    tb = 256
    n_tb = n // tb
    tc3 = tc.reshape(n_tb, 1, tb)
    idx3, dpos3 = pl.pallas_call(
        functools.partial(_argmin_body, kk=kk),
        grid=(n_tb,),
        in_specs=[
            pl.BlockSpec((tb, c), lambda i: (i, 0)),
            pl.BlockSpec((tb, c), lambda i: (i, 0)),
            pl.BlockSpec((kk, c), lambda i: (0, 0)),
            pl.BlockSpec((1, 1, tb), lambda i: (i, 0, 0)),
        ],
        out_specs=[
            pl.BlockSpec((1, 1, tb), lambda i: (i, 0, 0)),
            pl.BlockSpec((1, 1, tb), lambda i: (i, 0, 0)),
        ],
        out_shape=[
            jax.ShapeDtypeStruct((n_tb, 1, tb), jnp.int32),
            jax.ShapeDtypeStruct((n_tb, 1, tb), jnp.float32),
        ],
        scratch_shapes=[pltpu.VMEM((1, kk), jnp.float32)],
    )(z, t, codebook, tc3)
    return idx3.reshape(n), dpos3


def _sc_gather(codebook, idx):
    n = idx.shape[0]
    d = codebook.shape[1]
    info = plsc.get_sparse_core_info()
    nc, ns = info.num_cores, info.num_subcores
    nw = nc * ns
    b_per_w = n // nw
    mesh = plsc.VectorSubcoreMesh(core_axis_name="c", subcore_axis_name="s")

    @functools.partial(
        pl.kernel, mesh=mesh,
        out_type=jax.ShapeDtypeStruct((n, d), jnp.float32),
        scratch_types=[
            pltpu.VMEM((b_per_w,), jnp.int32),
            pltpu.VMEM((b_per_w, d), jnp.float32),
            pltpu.SemaphoreType.DMA,
        ],
    )
    def gather_k(table_hbm, idx_hbm, out_hbm, idx_v, rows_v, sem):
        wid = lax.axis_index("s") * nc + lax.axis_index("c")
        base = wid * b_per_w
        pltpu.sync_copy(idx_hbm.at[pl.ds(base, b_per_w)], idx_v)
        pltpu.async_copy(table_hbm.at[idx_v], rows_v, sem).wait()
        pltpu.sync_copy(rows_v, out_hbm.at[pl.ds(base, b_per_w)])

    return gather_k(codebook, idx)


def _loss_body(t_ref, n_ref, dp_ref, out_ref, *, n_total, n_blocks):
    i = pl.program_id(0)
    tb = t_ref[...]
    nb = n_ref[...]
    dn = jnp.sqrt(jnp.maximum(jnp.sum((tb - nb) ** 2, axis=1), 1e-12))
    dp = dp_ref[...].reshape(-1)
    losses = jnp.maximum(dp - dn + MARGIN_, 0.0)
    sat = (dn > dp + MARGIN_).astype(jnp.float32)
    lane = lax.broadcasted_iota(jnp.int32, (8, 128), 0)
    part = jnp.where(lane == 0, jnp.sum(losses),
           jnp.where(lane == 1, jnp.sum(dp),
           jnp.where(lane == 2, jnp.sum(dn),
           jnp.where(lane == 3, jnp.sum(sat), 0.0))))

    @pl.when(i == 0)
    def _():
        out_ref[...] = jnp.zeros_like(out_ref)

    out_ref[...] += part

    @pl.when(i == n_blocks - 1)
    def _():
        out_ref[...] = out_ref[...] / float(n_total)


def _triplet_stats(t, negs, dpos3):
    n, c = t.shape
    nb = 512
    n_blocks = n // nb
    dp2 = dpos3.reshape(n_blocks, 1, nb)
    out = pl.pallas_call(
        functools.partial(_loss_body, n_total=n, n_blocks=n_blocks),
        grid=(n_blocks,),
        in_specs=[
            pl.BlockSpec((nb, c), lambda i: (i, 0)),
            pl.BlockSpec((nb, c), lambda i: (i, 0)),
            pl.BlockSpec((1, 1, nb), lambda i: (i, 0, 0)),
        ],
        out_specs=pl.BlockSpec((8, 128), lambda i: (0, 0)),
        out_shape=jax.ShapeDtypeStruct((8, 128), jnp.float32),
    )(t, negs, dp2)
    return out[0, 0], out[1, 0], out[2, 0], out[3, 0]


def kernel(student_out, teacher_out, codebook, teacher_codes):
    b, c, t = student_out.shape
    n = b * t
    z = jnp.transpose(student_out, (0, 2, 1)).reshape(n, c)
    tt = jnp.transpose(teacher_out, (0, 2, 1)).reshape(n, c)
    tc = teacher_codes.astype(jnp.int32).reshape(n)

    idx, dpos3 = _mine_and_dpos(z, tt, codebook, tc)
    negs = _sc_gather(codebook, idx)
    loss, d_pos, d_neg, sat = _triplet_stats(tt, negs, dpos3)
    return (loss, d_pos, d_neg, sat)


# speedup vs baseline: 9.8660x; 1.0736x over previous
import functools

import jax
import jax.numpy as jnp
from jax import lax
from jax.experimental import pallas as pl
from jax.experimental.pallas import tpu as pltpu
from jax.experimental.pallas import tpu_sc as plsc

MARGIN_ = 0.5


def _argmin_body(z_ref, t_ref, cb_ref, tc_ref, idx_ref, dp_ref, c2_ref, *, kk):
    @pl.when(pl.program_id(0) == 0)
    def _():
        cb0 = cb_ref[...]
        ones = jnp.ones((1, cb0.shape[1]), jnp.float32)
        c2_ref[...] = lax.dot_general(ones, cb0 * cb0, (((1,), (1,)), ((), ())),
                                      preferred_element_type=jnp.float32)

    zb = z_ref[...]
    zc = lax.dot_general(zb * (-2.0), cb_ref[...], (((1,), (1,)), ((), ())),
                         preferred_element_type=jnp.float32)
    z2 = jnp.sum(zb * zb, axis=1, keepdims=True)
    e = jnp.maximum(zc + c2_ref[...], 1e-12 - z2)
    col = lax.broadcasted_iota(jnp.int32, e.shape, 1)
    tcb = tc_ref[...].reshape(-1, 1)
    em = jnp.where(col == tcb, jnp.inf, e)
    idx = jnp.argmin(em, axis=1).astype(jnp.int32)
    idx_ref[...] = idx.reshape(1, 1, -1)

    tb = t_ref[...]
    diff = tb - zb
    dp2 = jnp.sum(diff * diff, axis=1)
    dp_ref[...] = jnp.sqrt(jnp.maximum(dp2, 1e-12)).reshape(1, 1, -1)


def _mine_and_dpos(z, t, codebook, tc):
    n, c = z.shape
    kk = codebook.shape[0]
    tb = 512
    n_tb = n // tb
    tc3 = tc.reshape(n_tb, 1, tb)
    idx3, dpos3 = pl.pallas_call(
        functools.partial(_argmin_body, kk=kk),
        grid=(n_tb,),
        in_specs=[
            pl.BlockSpec((tb, c), lambda i: (i, 0)),
            pl.BlockSpec((tb, c), lambda i: (i, 0)),
            pl.BlockSpec((kk, c), lambda i: (0, 0)),
            pl.BlockSpec((1, 1, tb), lambda i: (i, 0, 0)),
        ],
        out_specs=[
            pl.BlockSpec((1, 1, tb), lambda i: (i, 0, 0)),
            pl.BlockSpec((1, 1, tb), lambda i: (i, 0, 0)),
        ],
        out_shape=[
            jax.ShapeDtypeStruct((n_tb, 1, tb), jnp.int32),
            jax.ShapeDtypeStruct((n_tb, 1, tb), jnp.float32),
        ],
        scratch_shapes=[pltpu.VMEM((1, kk), jnp.float32)],
    )(z, t, codebook, tc3)
    return idx3.reshape(n), dpos3


def _sc_gather(codebook, idx):
    n = idx.shape[0]
    d = codebook.shape[1]
    info = plsc.get_sparse_core_info()
    nc, ns = info.num_cores, info.num_subcores
    nw = nc * ns
    b_per_w = n // nw
    mesh = plsc.VectorSubcoreMesh(core_axis_name="c", subcore_axis_name="s")

    @functools.partial(
        pl.kernel, mesh=mesh,
        out_type=jax.ShapeDtypeStruct((n, d), jnp.float32),
        scratch_types=[
            pltpu.VMEM((b_per_w,), jnp.int32),
            pltpu.VMEM((b_per_w, d), jnp.float32),
            pltpu.SemaphoreType.DMA,
        ],
    )
    def gather_k(table_hbm, idx_hbm, out_hbm, idx_v, rows_v, sem):
        wid = lax.axis_index("s") * nc + lax.axis_index("c")
        base = wid * b_per_w
        pltpu.sync_copy(idx_hbm.at[pl.ds(base, b_per_w)], idx_v)
        pltpu.async_copy(table_hbm.at[idx_v], rows_v, sem).wait()
        pltpu.sync_copy(rows_v, out_hbm.at[pl.ds(base, b_per_w)])

    return gather_k(codebook, idx)


def _loss_body(t_ref, n_ref, dp_ref, out_ref, *, n_total, n_blocks):
    i = pl.program_id(0)
    tb = t_ref[...]
    nb = n_ref[...]
    dn = jnp.sqrt(jnp.maximum(jnp.sum((tb - nb) ** 2, axis=1), 1e-12))
    dp = dp_ref[...].reshape(-1)
    losses = jnp.maximum(dp - dn + MARGIN_, 0.0)
    sat = (dn > dp + MARGIN_).astype(jnp.float32)
    lane = lax.broadcasted_iota(jnp.int32, (8, 128), 0)
    part = jnp.where(lane == 0, jnp.sum(losses),
           jnp.where(lane == 1, jnp.sum(dp),
           jnp.where(lane == 2, jnp.sum(dn),
           jnp.where(lane == 3, jnp.sum(sat), 0.0))))

    @pl.when(i == 0)
    def _():
        out_ref[...] = jnp.zeros_like(out_ref)

    out_ref[...] += part

    @pl.when(i == n_blocks - 1)
    def _():
        out_ref[...] = out_ref[...] / float(n_total)


def _triplet_stats(t, negs, dpos3):
    n, c = t.shape
    nb = 512
    n_blocks = n // nb
    dp2 = dpos3.reshape(n_blocks, 1, nb)
    out = pl.pallas_call(
        functools.partial(_loss_body, n_total=n, n_blocks=n_blocks),
        grid=(n_blocks,),
        in_specs=[
            pl.BlockSpec((nb, c), lambda i: (i, 0)),
            pl.BlockSpec((nb, c), lambda i: (i, 0)),
            pl.BlockSpec((1, 1, nb), lambda i: (i, 0, 0)),
        ],
        out_specs=pl.BlockSpec((8, 128), lambda i: (0, 0)),
        out_shape=jax.ShapeDtypeStruct((8, 128), jnp.float32),
    )(t, negs, dp2)
    return out[0, 0], out[1, 0], out[2, 0], out[3, 0]


def kernel(student_out, teacher_out, codebook, teacher_codes):
    b, c, t = student_out.shape
    n = b * t
    z = jnp.transpose(student_out, (0, 2, 1)).reshape(n, c)
    tt = jnp.transpose(teacher_out, (0, 2, 1)).reshape(n, c)
    tc = teacher_codes.astype(jnp.int32).reshape(n)

    idx, dpos3 = _mine_and_dpos(z, tt, codebook, tc)
    negs = _sc_gather(codebook, idx)
    loss, d_pos, d_neg, sat = _triplet_stats(tt, negs, dpos3)
    return (loss, d_pos, d_neg, sat)


# speedup vs baseline: 9.9354x; 1.0070x over previous
import functools

import jax
import jax.numpy as jnp
from jax import lax
from jax.experimental import pallas as pl
from jax.experimental.pallas import tpu as pltpu
from jax.experimental.pallas import tpu_sc as plsc

MARGIN_ = 0.5


def _argmin_body(z_ref, t_ref, cb_ref, tc_ref, idx_ref, dp_ref, c2_ref, *, kk):
    @pl.when(pl.program_id(0) == 0)
    def _():
        cb0 = cb_ref[...]
        ones = jnp.ones((1, cb0.shape[1]), jnp.float32)
        c2_ref[...] = lax.dot_general(ones, cb0 * cb0, (((1,), (1,)), ((), ())),
                                      preferred_element_type=jnp.float32)

    zb = z_ref[...]
    zc = lax.dot_general(zb * (-2.0), cb_ref[...], (((1,), (1,)), ((), ())),
                         preferred_element_type=jnp.float32)
    z2 = jnp.sum(zb * zb, axis=1, keepdims=True)
    e = jnp.maximum(zc + c2_ref[...], 1e-12 - z2)
    col = lax.broadcasted_iota(jnp.int32, e.shape, 1)
    tcb = tc_ref[...].reshape(-1, 1)
    em = jnp.where(col == tcb, jnp.inf, e)
    idx = jnp.argmin(em, axis=1).astype(jnp.int32)
    idx_ref[...] = idx.reshape(1, 1, -1)

    tb = t_ref[...]
    diff = tb - zb
    dp2 = jnp.sum(diff * diff, axis=1)
    dp_ref[...] = jnp.sqrt(jnp.maximum(dp2, 1e-12)).reshape(1, 1, -1)


def _mine_and_dpos(z, t, codebook, tc):
    n, c = z.shape
    kk = codebook.shape[0]
    tb = 1152
    n_tb = n // tb
    tc3 = tc.reshape(n_tb, 1, tb)
    idx3, dpos3 = pl.pallas_call(
        functools.partial(_argmin_body, kk=kk),
        grid=(n_tb,),
        in_specs=[
            pl.BlockSpec((tb, c), lambda i: (i, 0)),
            pl.BlockSpec((tb, c), lambda i: (i, 0)),
            pl.BlockSpec((kk, c), lambda i: (0, 0)),
            pl.BlockSpec((1, 1, tb), lambda i: (i, 0, 0)),
        ],
        out_specs=[
            pl.BlockSpec((1, 1, tb), lambda i: (i, 0, 0)),
            pl.BlockSpec((1, 1, tb), lambda i: (i, 0, 0)),
        ],
        out_shape=[
            jax.ShapeDtypeStruct((n_tb, 1, tb), jnp.int32),
            jax.ShapeDtypeStruct((n_tb, 1, tb), jnp.float32),
        ],
        scratch_shapes=[pltpu.VMEM((1, kk), jnp.float32)],
    )(z, t, codebook, tc3)
    return idx3.reshape(n), dpos3


def _sc_gather(codebook, idx):
    n = idx.shape[0]
    d = codebook.shape[1]
    info = plsc.get_sparse_core_info()
    nc, ns = info.num_cores, info.num_subcores
    nw = nc * ns
    b_per_w = n // nw
    mesh = plsc.VectorSubcoreMesh(core_axis_name="c", subcore_axis_name="s")

    @functools.partial(
        pl.kernel, mesh=mesh,
        out_type=jax.ShapeDtypeStruct((n, d), jnp.float32),
        scratch_types=[
            pltpu.VMEM((b_per_w,), jnp.int32),
            pltpu.VMEM((b_per_w, d), jnp.float32),
            pltpu.SemaphoreType.DMA,
        ],
    )
    def gather_k(table_hbm, idx_hbm, out_hbm, idx_v, rows_v, sem):
        wid = lax.axis_index("s") * nc + lax.axis_index("c")
        base = wid * b_per_w
        pltpu.sync_copy(idx_hbm.at[pl.ds(base, b_per_w)], idx_v)
        pltpu.async_copy(table_hbm.at[idx_v], rows_v, sem).wait()
        pltpu.sync_copy(rows_v, out_hbm.at[pl.ds(base, b_per_w)])

    return gather_k(codebook, idx)


def _loss_body(t_ref, n_ref, dp_ref, out_ref, *, n_total, n_blocks):
    i = pl.program_id(0)
    tb = t_ref[...]
    nb = n_ref[...]
    dn = jnp.sqrt(jnp.maximum(jnp.sum((tb - nb) ** 2, axis=1), 1e-12))
    dp = dp_ref[...].reshape(-1)
    losses = jnp.maximum(dp - dn + MARGIN_, 0.0)
    sat = (dn > dp + MARGIN_).astype(jnp.float32)
    lane = lax.broadcasted_iota(jnp.int32, (8, 128), 0)
    part = jnp.where(lane == 0, jnp.sum(losses),
           jnp.where(lane == 1, jnp.sum(dp),
           jnp.where(lane == 2, jnp.sum(dn),
           jnp.where(lane == 3, jnp.sum(sat), 0.0))))

    @pl.when(i == 0)
    def _():
        out_ref[...] = jnp.zeros_like(out_ref)

    out_ref[...] += part

    @pl.when(i == n_blocks - 1)
    def _():
        out_ref[...] = out_ref[...] / float(n_total)


def _triplet_stats(t, negs, dpos3):
    n, c = t.shape
    nb = 512
    n_blocks = n // nb
    dp2 = dpos3.reshape(n_blocks, 1, nb)
    out = pl.pallas_call(
        functools.partial(_loss_body, n_total=n, n_blocks=n_blocks),
        grid=(n_blocks,),
        in_specs=[
            pl.BlockSpec((nb, c), lambda i: (i, 0)),
            pl.BlockSpec((nb, c), lambda i: (i, 0)),
            pl.BlockSpec((1, 1, nb), lambda i: (i, 0, 0)),
        ],
        out_specs=pl.BlockSpec((8, 128), lambda i: (0, 0)),
        out_shape=jax.ShapeDtypeStruct((8, 128), jnp.float32),
    )(t, negs, dp2)
    return out[0, 0], out[1, 0], out[2, 0], out[3, 0]


def kernel(student_out, teacher_out, codebook, teacher_codes):
    b, c, t = student_out.shape
    n = b * t
    z = jnp.transpose(student_out, (0, 2, 1)).reshape(n, c)
    tt = jnp.transpose(teacher_out, (0, 2, 1)).reshape(n, c)
    tc = teacher_codes.astype(jnp.int32).reshape(n)

    idx, dpos3 = _mine_and_dpos(z, tt, codebook, tc)
    negs = _sc_gather(codebook, idx)
    loss, d_pos, d_neg, sat = _triplet_stats(tt, negs, dpos3)
    return (loss, d_pos, d_neg, sat)
